# baseline (speedup 1.0000x reference)
import jax
import jax.numpy as jnp
from jax import lax
from jax.experimental import pallas as pl
from jax.experimental.pallas import tpu as pltpu
from jax.experimental.pallas import tpu_sc as plsc

_N = 10000
_E = 320000
_D = 128
_H = 128
_NUM_CLASSES = 10000

_NC = 2
_NS = 16
_NW = _NC * _NS
_EP = 327680
_EPT = _EP // _NW
_CH = 128
_NCHUNK = _EPT // _CH

_RS = 624
_RN = 640


def _prelu(v, a):
    return jnp.where(v >= 0, v, a * v)


def _scale_rows(rows_v, w_v):
    def grp(g, carry):
        wv = w_v[pl.ds(g * 16, 16)]
        for l in range(16):
            wsp = jnp.full((16,), wv[l])
            i = g * 16 + l
            for j in range(_H // 16):
                sl = pl.ds(j * 16, 16)
                rows_v[i, sl] = rows_v[i, sl] * wsp
        return carry

    lax.fori_loop(0, _CH // 16, grp, 0)


def _sc_body(x_hbm, src_hbm, dst_hbm, w_hbm, out_hbm,
             rows0, rows1, rows2, srcb0, srcb1, srcb2, dstb0, dstb1, dstb2,
             wb0, wb1, wb2, acc_sh,
             semi0, semi1, semi2, semd0, semd1, semd2,
             semr0, semr1, semr2, sems0, sems1, sems2):
    c = lax.axis_index("c")
    s = lax.axis_index("s")
    wid = s * _NC + c
    ebase = wid * _EPT
    rows = (rows0, rows1, rows2)
    srcb = (srcb0, srcb1, srcb2)
    dstb = (dstb0, dstb1, dstb2)
    wb = (wb0, wb1, wb2)
    semi = (semi0, semi1, semi2)
    semd = (semd0, semd1, semd2)
    semr = (semr0, semr1, semr2)
    sems = (sems0, sems1, sems2)

    def sw_load(k, m):
        off = ebase + k * _CH
        pltpu.async_copy(src_hbm.at[pl.ds(off, _CH)], srcb[m], semi[m])
        pltpu.async_copy(w_hbm.at[pl.ds(off, _CH)], wb[m], semi[m])

    def sw_wait(m):
        z = pl.ds(0, _CH)
        pltpu.make_async_copy(src_hbm.at[z], srcb[m], semi[m]).wait()
        pltpu.make_async_copy(w_hbm.at[z], wb[m], semi[m]).wait()

    def dst_load(k, m):
        off = ebase + k * _CH
        pltpu.async_copy(dst_hbm.at[pl.ds(off, _CH)], dstb[m], semd[m])

    def dst_wait(m):
        pltpu.make_async_copy(dst_hbm.at[pl.ds(0, _CH)], dstb[m],
                              semd[m]).wait()

    def gat_issue(k, m):
        pltpu.async_copy(x_hbm.at[srcb[m]], rows[m], semr[m])

    def gat_wait(m):
        pltpu.make_async_copy(x_hbm.at[srcb[m]], rows[m], semr[m]).wait()

    def scat_issue(m):
        pltpu.async_copy(rows[m], acc_sh.at[dstb[m]], sems[m], add=True)

    def scat_wait(m):
        pltpu.make_async_copy(rows[m], acc_sh.at[dstb[m]], sems[m]).wait()

    sw_load(0, 0)
    sw_load(1, 1)
    dst_load(0, 0)

    z16 = jnp.zeros((16,), jnp.float32)

    def zrow(r, carry):
        for j in range(_H // 16):
            rows1[r, pl.ds(j * 16, 16)] = z16
            rows2[r, pl.ds(j * 16, 16)] = z16
        return carry

    lax.fori_loop(0, _CH, zrow, 0)
    zi = jnp.zeros((16,), jnp.int32)
    for r in range(_CH // 16):
        dstb1[pl.ds(r * 16, 16)] = zi
        dstb2[pl.ds(r * 16, 16)] = zi
    for k in range(_RN // _CH):
        pltpu.sync_copy(rows1, acc_sh.at[pl.ds(s * _RS + k * _CH, _CH)])
    plsc.subcore_barrier()

    sw_wait(0)
    gat_issue(0, 0)
    pltpu.async_copy(rows1, acc_sh.at[dstb1], sems[1], add=True)
    pltpu.async_copy(rows2, acc_sh.at[dstb2], sems[2], add=True)

    def triple(p, carry):
        for b in range(3):
            k = 3 * p + b
            m = b
            n = (b + 1) % 3
            o = (b + 2) % 3
            sw_wait(n)
            gat_wait(m)
            scat_wait(n)
            gat_issue(k + 1, n)
            dst_load(k + 1, n)
            sw_load(k + 2, o)
            _scale_rows(rows[m], wb[m])
            dst_wait(m)
            scat_issue(m)
        return carry

    lax.fori_loop(0, (_NCHUNK - 2) // 3, triple, 0)

    sw_wait(1)
    gat_wait(0)
    scat_wait(1)
    gat_issue(_NCHUNK - 1, 1)
    dst_load(_NCHUNK - 1, 1)
    _scale_rows(rows0, wb0)
    dst_wait(0)
    scat_issue(0)

    gat_wait(1)
    scat_wait(2)
    _scale_rows(rows1, wb1)
    dst_wait(1)
    scat_issue(1)

    scat_wait(0)
    scat_wait(1)

    plsc.subcore_barrier()
    pltpu.sync_copy(acc_sh.at[pl.ds(s * _RS, _RN)],
                    out_hbm.at[pl.ds(c * _N + s * _RS, _RN)])


def _sc_scatter(x1, src, dst, w):
    f = pl.kernel(
        _sc_body,
        out_type=jax.ShapeDtypeStruct((2 * _N, _H), jnp.float32),
        mesh=plsc.VectorSubcoreMesh(core_axis_name="c", subcore_axis_name="s"),
        scratch_types=[
            pltpu.VMEM((_CH, _H), jnp.float32),
            pltpu.VMEM((_CH, _H), jnp.float32),
            pltpu.VMEM((_CH, _H), jnp.float32),
            pltpu.VMEM((_CH,), jnp.int32),
            pltpu.VMEM((_CH,), jnp.int32),
            pltpu.VMEM((_CH,), jnp.int32),
            pltpu.VMEM((_CH,), jnp.int32),
            pltpu.VMEM((_CH,), jnp.int32),
            pltpu.VMEM((_CH,), jnp.int32),
            pltpu.VMEM((_CH,), jnp.float32),
            pltpu.VMEM((_CH,), jnp.float32),
            pltpu.VMEM((_CH,), jnp.float32),
            pltpu.VMEM_SHARED((_N, _H), jnp.float32),
        ] + [pltpu.SemaphoreType.DMA] * 12,
    )
    return f(x1, src, dst, w)


def _tc_pre_body(x_ref, w_ref, b_ref, a_ref, o_ref):
    o = jnp.dot(x_ref[...], w_ref[...], preferred_element_type=jnp.float32)
    o_ref[...] = _prelu(o + b_ref[...], a_ref[0, 0])


def _tc_pre(x, w, b, a):
    return pl.pallas_call(
        _tc_pre_body,
        out_shape=jax.ShapeDtypeStruct((_N, _H), jnp.float32),
    )(x, w, b.reshape(1, _H), a.reshape(1, 1))


def _tc_layer_body(p_ref, x1_ref, relw_ref, relb_ref, rootw_ref, a_ref, o_ref):
    agg = p_ref[0] + p_ref[1]
    o = (jnp.dot(agg, relw_ref[...], preferred_element_type=jnp.float32)
         + relb_ref[...]
         + jnp.dot(x1_ref[...], rootw_ref[...],
                   preferred_element_type=jnp.float32))
    o_ref[...] = _prelu(o, a_ref[0, 0])


def _tc_layer(parts, x1, relw, relb, rootw, a):
    return pl.pallas_call(
        _tc_layer_body,
        out_shape=jax.ShapeDtypeStruct((_N, _H), jnp.float32),
    )(parts.reshape(2, _N, _H), x1, relw, relb.reshape(1, _H), rootw,
      a.reshape(1, 1))


def _tc_post_body(x_ref, x1_ref, postw_ref, postb_ref, posta_ref,
                  finwx_ref, finwh_ref, finb_ref, fina_ref, batch_ref, o_ref):
    h = _prelu(jnp.dot(x1_ref[...], postw_ref[...],
                       preferred_element_type=jnp.float32) + postb_ref[...],
               posta_ref[0, 0])
    f = (jnp.dot(x_ref[...], finwx_ref[...],
                 preferred_element_type=jnp.float32)
         + jnp.dot(h, finwh_ref[...], preferred_element_type=jnp.float32)
         + finb_ref[0, 0])
    f = _prelu(f, fina_ref[0, 0]) + batch_ref[...]
    m = jnp.max(f)
    lse = jnp.log(jnp.sum(jnp.exp(f - m))) + m
    o_ref[...] = f - lse


def _tc_post(x, x1, postw, postb, posta, finw, finb, fina, batch):
    return pl.pallas_call(
        _tc_post_body,
        out_shape=jax.ShapeDtypeStruct((_N, 1), jnp.float32),
    )(x, x1, postw, postb.reshape(1, _H), posta.reshape(1, 1),
      finw[:_D], finw[_D:], finb.reshape(1, 1), fina.reshape(1, 1),
      batch.astype(jnp.float32).reshape(_N, 1))


def kernel(x, edge_index, edge_weights, batch, pre_W, pre_b, pre_a,
           rel_W0, rel_b0, root_W0, a0,
           rel_W1, rel_b1, root_W1, a1,
           rel_W2, rel_b2, root_W2, a2,
           post_W, post_b, post_a, fin_W, fin_b, fin_a):
    pad = _EP - _E
    src = jnp.concatenate([edge_index[0], jnp.zeros((pad,), jnp.int32)])
    dst = jnp.concatenate([edge_index[1], jnp.zeros((pad,), jnp.int32)])
    ew = jnp.concatenate([edge_weights, jnp.zeros((pad,), jnp.float32)])
    x1 = _tc_pre(x, pre_W, pre_b, pre_a)
    for relw, relb, rootw, a in ((rel_W0, rel_b0, root_W0, a0),
                                 (rel_W1, rel_b1, root_W1, a1),
                                 (rel_W2, rel_b2, root_W2, a2)):
        parts = _sc_scatter(x1, src, dst, ew)
        x1 = _tc_layer(parts, x1, relw, relb, rootw, a)
    out = _tc_post(x, x1, post_W, post_b, post_a, fin_W, fin_b, fin_a, batch)
    return out.reshape(1, _NUM_CLASSES)


# speedup vs baseline: 1.5975x; 1.5975x over previous
import jax
import jax.numpy as jnp
from jax import lax
from jax.experimental import pallas as pl
from jax.experimental.pallas import tpu as pltpu
from jax.experimental.pallas import tpu_sc as plsc

_N = 10000
_E = 320000
_D = 128
_H = 128
_NUM_CLASSES = 10000

_NC = 2
_NS = 16
_NW = _NC * _NS
_EP = 323584
_EPT = _EP // _NW
_CH = 64
_NCHUNK = _EPT // _CH

_RS = 624
_RN = 640


def _prelu(v, a):
    return jnp.where(v >= 0, v, a * v)


def _scale_rows(rows_v, w_v):
    def grp(g, carry):
        wv = w_v[pl.ds(g * 16, 16)]
        for l in range(16):
            wsp = jnp.full((16,), wv[l])
            i = g * 16 + l
            for j in range(_H // 16):
                sl = pl.ds(j * 16, 16)
                rows_v[i, sl] = rows_v[i, sl] * wsp
        return carry

    lax.fori_loop(0, _CH // 16, grp, 0)


def _sc_body(x_hbm, src_hbm, dst_hbm, w_hbm, out_hbm,
             rows0, rows1, rows2, srcb0, srcb1, srcb2, dstb0, dstb1, dstb2,
             wb0, wb1, wb2, acc_sh,
             semi0, semi1, semi2, semd0, semd1, semd2,
             semr0, semr1, semr2, sems0, sems1, sems2):
    c = lax.axis_index("c")
    s = lax.axis_index("s")
    wid = s * _NC + c
    ebase = wid * _EPT
    rows = (rows0, rows1, rows2)
    srcb = (srcb0, srcb1, srcb2)
    dstb = (dstb0, dstb1, dstb2)
    wb = (wb0, wb1, wb2)
    semi = (semi0, semi1, semi2)
    semd = (semd0, semd1, semd2)
    semr = (semr0, semr1, semr2)
    sems = (sems0, sems1, sems2)

    def sw_load(k, m):
        off = ebase + k * _CH
        pltpu.async_copy(src_hbm.at[pl.ds(off, _CH)], srcb[m], semi[m])
        pltpu.async_copy(w_hbm.at[pl.ds(off, _CH)], wb[m], semi[m])

    def sw_wait(m):
        z = pl.ds(0, _CH)
        pltpu.make_async_copy(src_hbm.at[z], srcb[m], semi[m]).wait()
        pltpu.make_async_copy(w_hbm.at[z], wb[m], semi[m]).wait()

    def dst_load(k, m):
        off = ebase + k * _CH
        pltpu.async_copy(dst_hbm.at[pl.ds(off, _CH)], dstb[m], semd[m])

    def dst_wait(m):
        pltpu.make_async_copy(dst_hbm.at[pl.ds(0, _CH)], dstb[m],
                              semd[m]).wait()

    def gat_issue(k, m):
        pltpu.async_copy(x_hbm.at[srcb[m]], rows[m], semr[m])

    def gat_wait(m):
        pltpu.make_async_copy(x_hbm.at[srcb[m]], rows[m], semr[m]).wait()

    def scat_issue(m):
        pltpu.async_copy(rows[m], acc_sh.at[dstb[m]], sems[m], add=True)

    def scat_wait(m):
        pltpu.make_async_copy(rows[m], acc_sh.at[dstb[m]], sems[m]).wait()

    sw_load(0, 0)
    sw_load(1, 1)
    dst_load(0, 0)

    z16 = jnp.zeros((16,), jnp.float32)

    def zrow(r, carry):
        for j in range(_H // 16):
            rows1[r, pl.ds(j * 16, 16)] = z16
            rows2[r, pl.ds(j * 16, 16)] = z16
        return carry

    lax.fori_loop(0, _CH, zrow, 0)
    zi = jnp.zeros((16,), jnp.int32)
    for r in range(_CH // 16):
        dstb1[pl.ds(r * 16, 16)] = zi
        dstb2[pl.ds(r * 16, 16)] = zi
    for k in range(_RN // _CH):
        pltpu.sync_copy(rows1, acc_sh.at[pl.ds(s * _RS + k * _CH, _CH)])
    plsc.subcore_barrier()

    sw_wait(0)
    gat_issue(0, 0)
    pltpu.async_copy(rows1, acc_sh.at[dstb1], sems[1], add=True)
    pltpu.async_copy(rows2, acc_sh.at[dstb2], sems[2], add=True)

    def triple(p, carry):
        for b in range(3):
            k = 3 * p + b
            m = b
            n = (b + 1) % 3
            o = (b + 2) % 3
            sw_wait(n)
            gat_wait(m)
            scat_wait(n)
            gat_issue(k + 1, n)
            dst_load(k + 1, n)
            sw_load(k + 2, o)
            _scale_rows(rows[m], wb[m])
            dst_wait(m)
            scat_issue(m)
        return carry

    lax.fori_loop(0, (_NCHUNK - 2) // 3, triple, 0)

    sw_wait(1)
    gat_wait(0)
    scat_wait(1)
    gat_issue(_NCHUNK - 1, 1)
    dst_load(_NCHUNK - 1, 1)
    _scale_rows(rows0, wb0)
    dst_wait(0)
    scat_issue(0)

    gat_wait(1)
    scat_wait(2)
    _scale_rows(rows1, wb1)
    dst_wait(1)
    scat_issue(1)

    scat_wait(0)
    scat_wait(1)

    plsc.subcore_barrier()
    pltpu.sync_copy(acc_sh.at[pl.ds(s * _RS, _RN)],
                    out_hbm.at[pl.ds(c * _N + s * _RS, _RN)])


def _sc_scatter(x1, src, dst, w):
    f = pl.kernel(
        _sc_body,
        out_type=jax.ShapeDtypeStruct((2 * _N, _H), jnp.float32),
        mesh=plsc.VectorSubcoreMesh(core_axis_name="c", subcore_axis_name="s"),
        scratch_types=[
            pltpu.VMEM((_CH, _H), jnp.float32),
            pltpu.VMEM((_CH, _H), jnp.float32),
            pltpu.VMEM((_CH, _H), jnp.float32),
            pltpu.VMEM((_CH,), jnp.int32),
            pltpu.VMEM((_CH,), jnp.int32),
            pltpu.VMEM((_CH,), jnp.int32),
            pltpu.VMEM((_CH,), jnp.int32),
            pltpu.VMEM((_CH,), jnp.int32),
            pltpu.VMEM((_CH,), jnp.int32),
            pltpu.VMEM((_CH,), jnp.float32),
            pltpu.VMEM((_CH,), jnp.float32),
            pltpu.VMEM((_CH,), jnp.float32),
            pltpu.VMEM_SHARED((_N, _H), jnp.float32),
        ] + [pltpu.SemaphoreType.DMA] * 12,
    )
    return f(x1, src, dst, w)


def _tc_pre_body(x_ref, w_ref, b_ref, a_ref, o_ref):
    o = jnp.dot(x_ref[...], w_ref[...], preferred_element_type=jnp.float32)
    o_ref[...] = _prelu(o + b_ref[...], a_ref[0, 0])


def _tc_pre(x, w, b, a):
    return pl.pallas_call(
        _tc_pre_body,
        out_shape=jax.ShapeDtypeStruct((_N, _H), jnp.float32),
    )(x, w, b.reshape(1, _H), a.reshape(1, 1))


def _tc_layer_body(p_ref, x1_ref, relw_ref, relb_ref, rootw_ref, a_ref, o_ref):
    agg = p_ref[0] + p_ref[1]
    o = (jnp.dot(agg, relw_ref[...], preferred_element_type=jnp.float32)
         + relb_ref[...]
         + jnp.dot(x1_ref[...], rootw_ref[...],
                   preferred_element_type=jnp.float32))
    o_ref[...] = _prelu(o, a_ref[0, 0])


def _tc_layer(parts, x1, relw, relb, rootw, a):
    return pl.pallas_call(
        _tc_layer_body,
        out_shape=jax.ShapeDtypeStruct((_N, _H), jnp.float32),
    )(parts.reshape(2, _N, _H), x1, relw, relb.reshape(1, _H), rootw,
      a.reshape(1, 1))


def _tc_post_body(x_ref, x1_ref, postw_ref, postb_ref, posta_ref,
                  finwx_ref, finwh_ref, finb_ref, fina_ref, batch_ref, o_ref):
    h = _prelu(jnp.dot(x1_ref[...], postw_ref[...],
                       preferred_element_type=jnp.float32) + postb_ref[...],
               posta_ref[0, 0])
    f = (jnp.dot(x_ref[...], finwx_ref[...],
                 preferred_element_type=jnp.float32)
         + jnp.dot(h, finwh_ref[...], preferred_element_type=jnp.float32)
         + finb_ref[0, 0])
    f = _prelu(f, fina_ref[0, 0]) + batch_ref[...]
    m = jnp.max(f)
    lse = jnp.log(jnp.sum(jnp.exp(f - m))) + m
    o_ref[...] = f - lse


def _tc_post(x, x1, postw, postb, posta, finw, finb, fina, batch):
    return pl.pallas_call(
        _tc_post_body,
        out_shape=jax.ShapeDtypeStruct((_N, 1), jnp.float32),
    )(x, x1, postw, postb.reshape(1, _H), posta.reshape(1, 1),
      finw[:_D], finw[_D:], finb.reshape(1, 1), fina.reshape(1, 1),
      batch.astype(jnp.float32).reshape(_N, 1))


def kernel(x, edge_index, edge_weights, batch, pre_W, pre_b, pre_a,
           rel_W0, rel_b0, root_W0, a0,
           rel_W1, rel_b1, root_W1, a1,
           rel_W2, rel_b2, root_W2, a2,
           post_W, post_b, post_a, fin_W, fin_b, fin_a):
    pad = _EP - _E
    src = jnp.concatenate([edge_index[0], jnp.zeros((pad,), jnp.int32)])
    dst = jnp.concatenate([edge_index[1], jnp.zeros((pad,), jnp.int32)])
    ew = jnp.concatenate([edge_weights, jnp.zeros((pad,), jnp.float32)])
    x1 = _tc_pre(x, pre_W, pre_b, pre_a)
    for relw, relb, rootw, a in ((rel_W0, rel_b0, root_W0, a0),
                                 (rel_W1, rel_b1, root_W1, a1),
                                 (rel_W2, rel_b2, root_W2, a2)):
        parts = _sc_scatter(x1, src, dst, ew)
        x1 = _tc_layer(parts, x1, relw, relb, rootw, a)
    out = _tc_post(x, x1, post_W, post_b, post_a, fin_W, fin_b, fin_a, batch)
    return out.reshape(1, _NUM_CLASSES)


# speedup vs baseline: 2.8953x; 1.8124x over previous
import jax
import jax.numpy as jnp
from jax import lax
from jax.experimental import pallas as pl
from jax.experimental.pallas import tpu as pltpu
from jax.experimental.pallas import tpu_sc as plsc

_N = 10000
_E = 320000
_D = 128
_H = 128
_NUM_CLASSES = 10000

_NC = 2
_NS = 16
_NW = _NC * _NS
_EP = 323584
_EPT = _EP // _NW
_CH = 64
_NCHUNK = _EPT // _CH

_RS = 624
_RN = 640


def _prelu(v, a):
    return jnp.where(v >= 0, v, a * v)


def _scale_rows(rows_v, w_v):
    def grp(g, carry):
        wv = w_v[pl.ds(g * 16, 16)]
        for l in range(16):
            wsp = jnp.full((16,), wv[l])
            i = g * 16 + l
            for j in range(_H // 16):
                sl = pl.ds(j * 16, 16)
                rows_v[i, sl] = rows_v[i, sl] * wsp
        return carry

    lax.fori_loop(0, _CH // 16, grp, 0)


def _sc_body(x_hbm, src_hbm, dst_hbm, w_hbm, out_hbm,
             rows0, rows1, rows2, srcb0, srcb1, srcb2, dstb0, dstb1, dstb2,
             wb0, wb1, wb2, acc_sh,
             semi0, semi1, semi2, semd0, semd1, semd2,
             semr0, semr1, semr2, sems0, sems1, sems2):
    c = lax.axis_index("c")
    s = lax.axis_index("s")
    wid = s * _NC + c
    ebase = wid * _EPT
    rows = (rows0, rows1, rows2)
    srcb = (srcb0, srcb1, srcb2)
    dstb = (dstb0, dstb1, dstb2)
    wb = (wb0, wb1, wb2)
    semi = (semi0, semi1, semi2)
    semd = (semd0, semd1, semd2)
    semr = (semr0, semr1, semr2)
    sems = (sems0, sems1, sems2)

    def sw_load(k, m):
        off = ebase + k * _CH
        pltpu.async_copy(src_hbm.at[pl.ds(off, _CH)], srcb[m], semi[m])
        pltpu.async_copy(w_hbm.at[pl.ds(off, _CH)], wb[m], semi[m])

    def sw_wait(m):
        z = pl.ds(0, _CH)
        pltpu.make_async_copy(src_hbm.at[z], srcb[m], semi[m]).wait()
        pltpu.make_async_copy(w_hbm.at[z], wb[m], semi[m]).wait()

    def dst_load(k, m):
        off = ebase + k * _CH
        pltpu.async_copy(dst_hbm.at[pl.ds(off, _CH)], dstb[m], semd[m])

    def dst_wait(m):
        pltpu.make_async_copy(dst_hbm.at[pl.ds(0, _CH)], dstb[m],
                              semd[m]).wait()

    def gat_issue(k, m):
        pltpu.async_copy(x_hbm.at[srcb[m]], rows[m], semr[m])

    def gat_wait(m):
        pltpu.make_async_copy(x_hbm.at[srcb[m]], rows[m], semr[m]).wait()

    def scat_issue(m):
        pltpu.async_copy(rows[m], acc_sh.at[dstb[m]], sems[m], add=True)

    def scat_wait(m):
        pltpu.make_async_copy(rows[m], acc_sh.at[dstb[m]], sems[m]).wait()

    sw_load(0, 0)
    sw_load(1, 1)
    dst_load(0, 0)

    z16 = jnp.zeros((16,), jnp.float32)

    def zrow(r, carry):
        for j in range(_H // 16):
            rows1[r, pl.ds(j * 16, 16)] = z16
            rows2[r, pl.ds(j * 16, 16)] = z16
        return carry

    lax.fori_loop(0, _CH, zrow, 0)
    zi = jnp.zeros((16,), jnp.int32)
    for r in range(_CH // 16):
        dstb1[pl.ds(r * 16, 16)] = zi
        dstb2[pl.ds(r * 16, 16)] = zi
    for k in range(_RN // _CH):
        pltpu.sync_copy(rows1, acc_sh.at[pl.ds(s * _RS + k * _CH, _CH)])
    plsc.subcore_barrier()

    sw_wait(0)
    gat_issue(0, 0)
    pltpu.async_copy(rows1, acc_sh.at[dstb1], sems[1], add=True)
    pltpu.async_copy(rows2, acc_sh.at[dstb2], sems[2], add=True)

    def triple(p, carry):
        for b in range(3):
            k = 3 * p + b
            m = b
            n = (b + 1) % 3
            o = (b + 2) % 3
            sw_wait(n)
            gat_wait(m)
            scat_wait(n)
            gat_issue(k + 1, n)
            dst_load(k + 1, n)
            sw_load(k + 2, o)
            _scale_rows(rows[m], wb[m])
            dst_wait(m)
            scat_issue(m)
        return carry

    lax.fori_loop(0, (_NCHUNK - 2) // 3, triple, 0)

    sw_wait(1)
    gat_wait(0)
    scat_wait(1)
    gat_issue(_NCHUNK - 1, 1)
    dst_load(_NCHUNK - 1, 1)
    _scale_rows(rows0, wb0)
    dst_wait(0)
    scat_issue(0)

    gat_wait(1)
    scat_wait(2)
    _scale_rows(rows1, wb1)
    dst_wait(1)
    scat_issue(1)

    scat_wait(0)
    scat_wait(1)

    plsc.subcore_barrier()
    pltpu.sync_copy(acc_sh.at[pl.ds(s * _RS, _RN)],
                    out_hbm.at[pl.ds(c * _N + s * _RS, _RN)])


def _sc_scatter(x1, src, dst, w):
    f = pl.kernel(
        _sc_body,
        out_type=jax.ShapeDtypeStruct((2 * _N, _H), jnp.float32),
        mesh=plsc.VectorSubcoreMesh(core_axis_name="c", subcore_axis_name="s"),
        scratch_types=[
            pltpu.VMEM((_CH, _H), jnp.float32),
            pltpu.VMEM((_CH, _H), jnp.float32),
            pltpu.VMEM((_CH, _H), jnp.float32),
            pltpu.VMEM((_CH,), jnp.int32),
            pltpu.VMEM((_CH,), jnp.int32),
            pltpu.VMEM((_CH,), jnp.int32),
            pltpu.VMEM((_CH,), jnp.int32),
            pltpu.VMEM((_CH,), jnp.int32),
            pltpu.VMEM((_CH,), jnp.int32),
            pltpu.VMEM((_CH,), jnp.float32),
            pltpu.VMEM((_CH,), jnp.float32),
            pltpu.VMEM((_CH,), jnp.float32),
            pltpu.VMEM_SHARED((_N, _H), jnp.float32),
        ] + [pltpu.SemaphoreType.DMA] * 12,
    )
    return f(x1, src, dst, w)


def _tc_pre_body(x_ref, w_ref, b_ref, a_ref, o_ref):
    o = jnp.dot(x_ref[...], w_ref[...], preferred_element_type=jnp.float32)
    o_ref[...] = _prelu(o + b_ref[...], a_ref[0, 0])


def _tc_pre(x, w, b, a):
    return pl.pallas_call(
        _tc_pre_body,
        out_shape=jax.ShapeDtypeStruct((_N, _H), jnp.float32),
    )(x, w, b.reshape(1, _H), a.reshape(1, 1))


def _tc_layer_body(p_ref, x1_ref, relw_ref, relb_ref, rootw_ref, a_ref, o_ref):
    agg = p_ref[0] + p_ref[1]
    o = (jnp.dot(agg, relw_ref[...], preferred_element_type=jnp.float32)
         + relb_ref[...]
         + jnp.dot(x1_ref[...], rootw_ref[...],
                   preferred_element_type=jnp.float32))
    o_ref[...] = _prelu(o, a_ref[0, 0])


def _tc_layer(parts, x1, relw, relb, rootw, a):
    return pl.pallas_call(
        _tc_layer_body,
        out_shape=jax.ShapeDtypeStruct((_N, _H), jnp.float32),
    )(parts.reshape(2, _N, _H), x1, relw, relb.reshape(1, _H), rootw,
      a.reshape(1, 1))


def _tc_post_body(x_ref, x1_ref, postw_ref, postb_ref, posta_ref,
                  finwx_ref, finwh_ref, finb_ref, fina_ref, batch_ref, o_ref):
    h = _prelu(jnp.dot(x1_ref[...], postw_ref[...],
                       preferred_element_type=jnp.float32) + postb_ref[...],
               posta_ref[0, 0])
    f = (jnp.dot(x_ref[...], finwx_ref[...],
                 preferred_element_type=jnp.float32)
         + jnp.dot(h, finwh_ref[...], preferred_element_type=jnp.float32)
         + finb_ref[0, 0])
    f = _prelu(f, fina_ref[0, 0]) + batch_ref[...]
    m = jnp.max(f)
    lse = jnp.log(jnp.sum(jnp.exp(f - m))) + m
    o_ref[...] = f - lse


def _tc_post(x, x1, postw, postb, posta, finw, finb, fina, batch):
    return pl.pallas_call(
        _tc_post_body,
        out_shape=jax.ShapeDtypeStruct((_N, 1), jnp.float32),
    )(x, x1, postw, postb.reshape(1, _H), posta.reshape(1, 1),
      finw[:_D], finw[_D:], finb.reshape(1, 1), fina.reshape(1, 1),
      batch.astype(jnp.float32).reshape(_N, 1))


def kernel(x, edge_index, edge_weights, batch, pre_W, pre_b, pre_a,
           rel_W0, rel_b0, root_W0, a0,
           rel_W1, rel_b1, root_W1, a1,
           rel_W2, rel_b2, root_W2, a2,
           post_W, post_b, post_a, fin_W, fin_b, fin_a):
    pad = _EP - _E
    spread = (jnp.arange(pad, dtype=jnp.int32) * 13) % _N
    src = jnp.concatenate([edge_index[0], spread])
    dst = jnp.concatenate([edge_index[1], spread])
    ew = jnp.concatenate([edge_weights, jnp.zeros((pad,), jnp.float32)])
    x1 = _tc_pre(x, pre_W, pre_b, pre_a)
    for relw, relb, rootw, a in ((rel_W0, rel_b0, root_W0, a0),
                                 (rel_W1, rel_b1, root_W1, a1),
                                 (rel_W2, rel_b2, root_W2, a2)):
        parts = _sc_scatter(x1, src, dst, ew)
        x1 = _tc_layer(parts, x1, relw, relb, rootw, a)
    out = _tc_post(x, x1, post_W, post_b, post_a, fin_W, fin_b, fin_a, batch)
    return out.reshape(1, _NUM_CLASSES)


# speedup vs baseline: 3.5579x; 1.2289x over previous
import jax
import jax.numpy as jnp
from jax import lax
from jax.experimental import pallas as pl
from jax.experimental.pallas import tpu as pltpu
from jax.experimental.pallas import tpu_sc as plsc

_N = 10000
_E = 320000
_D = 128
_H = 128
_NUM_CLASSES = 10000

_NC = 2
_NS = 16
_NW = _NC * _NS
_EP = 327680
_EPT = _EP // _NW
_CH = 128
_NCHUNK = _EPT // _CH

_RS = 624
_RN = 640


def _prelu(v, a):
    return jnp.where(v >= 0, v, a * v)


def _scale_rows(rows_v, w_v):
    def grp(g, carry):
        wv = w_v[pl.ds(g * 16, 16)]
        for l in range(16):
            wsp = jnp.full((16,), wv[l])
            i = g * 16 + l
            for j in range(_H // 16):
                sl = pl.ds(j * 16, 16)
                rows_v[i, sl] = rows_v[i, sl] * wsp
        return carry

    lax.fori_loop(0, _CH // 16, grp, 0)


def _sc_body(x_hbm, src_hbm, dst_hbm, w_hbm, out_hbm,
             rows0, rows1, rows2, srcb0, srcb1, srcb2, dstb0, dstb1, dstb2,
             wb0, wb1, wb2, acc_sh,
             semi0, semi1, semi2, semd0, semd1, semd2,
             semr0, semr1, semr2, sems0, sems1, sems2):
    c = lax.axis_index("c")
    s = lax.axis_index("s")
    wid = s * _NC + c
    ebase = wid * _EPT
    rows = (rows0, rows1, rows2)
    srcb = (srcb0, srcb1, srcb2)
    dstb = (dstb0, dstb1, dstb2)
    wb = (wb0, wb1, wb2)
    semi = (semi0, semi1, semi2)
    semd = (semd0, semd1, semd2)
    semr = (semr0, semr1, semr2)
    sems = (sems0, sems1, sems2)

    def sw_load(k, m):
        off = ebase + k * _CH
        pltpu.async_copy(src_hbm.at[pl.ds(off, _CH)], srcb[m], semi[m])
        pltpu.async_copy(w_hbm.at[pl.ds(off, _CH)], wb[m], semi[m])

    def sw_wait(m):
        z = pl.ds(0, _CH)
        pltpu.make_async_copy(src_hbm.at[z], srcb[m], semi[m]).wait()
        pltpu.make_async_copy(w_hbm.at[z], wb[m], semi[m]).wait()

    def dst_load(k, m):
        off = ebase + k * _CH
        pltpu.async_copy(dst_hbm.at[pl.ds(off, _CH)], dstb[m], semd[m])

    def dst_wait(m):
        pltpu.make_async_copy(dst_hbm.at[pl.ds(0, _CH)], dstb[m],
                              semd[m]).wait()

    def gat_issue(k, m):
        pltpu.async_copy(x_hbm.at[srcb[m]], rows[m], semr[m])

    def gat_wait(m):
        pltpu.make_async_copy(x_hbm.at[srcb[m]], rows[m], semr[m]).wait()

    def scat_issue(m):
        pltpu.async_copy(rows[m], acc_sh.at[dstb[m]], sems[m], add=True)

    def scat_wait(m):
        pltpu.make_async_copy(rows[m], acc_sh.at[dstb[m]], sems[m]).wait()

    sw_load(0, 0)
    sw_load(1, 1)
    dst_load(0, 0)

    z16 = jnp.zeros((16,), jnp.float32)

    def zrow(r, carry):
        for j in range(_H // 16):
            rows1[r, pl.ds(j * 16, 16)] = z16
            rows2[r, pl.ds(j * 16, 16)] = z16
        return carry

    lax.fori_loop(0, _CH, zrow, 0)
    zi = jnp.zeros((16,), jnp.int32)
    for r in range(_CH // 16):
        dstb1[pl.ds(r * 16, 16)] = zi
        dstb2[pl.ds(r * 16, 16)] = zi
    for k in range(_RN // _CH):
        pltpu.sync_copy(rows1, acc_sh.at[pl.ds(s * _RS + k * _CH, _CH)])
    plsc.subcore_barrier()

    sw_wait(0)
    gat_issue(0, 0)
    pltpu.async_copy(rows1, acc_sh.at[dstb1], sems[1], add=True)
    pltpu.async_copy(rows2, acc_sh.at[dstb2], sems[2], add=True)

    def triple(p, carry):
        for b in range(3):
            k = 3 * p + b
            m = b
            n = (b + 1) % 3
            o = (b + 2) % 3
            sw_wait(n)
            gat_wait(m)
            scat_wait(n)
            gat_issue(k + 1, n)
            dst_load(k + 1, n)
            sw_load(k + 2, o)
            _scale_rows(rows[m], wb[m])
            dst_wait(m)
            scat_issue(m)
        return carry

    lax.fori_loop(0, (_NCHUNK - 2) // 3, triple, 0)

    sw_wait(1)
    gat_wait(0)
    scat_wait(1)
    gat_issue(_NCHUNK - 1, 1)
    dst_load(_NCHUNK - 1, 1)
    _scale_rows(rows0, wb0)
    dst_wait(0)
    scat_issue(0)

    gat_wait(1)
    scat_wait(2)
    _scale_rows(rows1, wb1)
    dst_wait(1)
    scat_issue(1)

    scat_wait(0)
    scat_wait(1)

    plsc.subcore_barrier()
    pltpu.sync_copy(acc_sh.at[pl.ds(s * _RS, _RN)],
                    out_hbm.at[pl.ds(c * _N + s * _RS, _RN)])


def _sc_scatter(x1, src, dst, w):
    f = pl.kernel(
        _sc_body,
        out_type=jax.ShapeDtypeStruct((2 * _N, _H), jnp.float32),
        mesh=plsc.VectorSubcoreMesh(core_axis_name="c", subcore_axis_name="s"),
        scratch_types=[
            pltpu.VMEM((_CH, _H), jnp.float32),
            pltpu.VMEM((_CH, _H), jnp.float32),
            pltpu.VMEM((_CH, _H), jnp.float32),
            pltpu.VMEM((_CH,), jnp.int32),
            pltpu.VMEM((_CH,), jnp.int32),
            pltpu.VMEM((_CH,), jnp.int32),
            pltpu.VMEM((_CH,), jnp.int32),
            pltpu.VMEM((_CH,), jnp.int32),
            pltpu.VMEM((_CH,), jnp.int32),
            pltpu.VMEM((_CH,), jnp.float32),
            pltpu.VMEM((_CH,), jnp.float32),
            pltpu.VMEM((_CH,), jnp.float32),
            pltpu.VMEM_SHARED((_N, _H), jnp.float32),
        ] + [pltpu.SemaphoreType.DMA] * 12,
    )
    return f(x1, src, dst, w)


def _tc_pre_body(x_ref, w_ref, b_ref, a_ref, o_ref):
    o = jnp.dot(x_ref[...], w_ref[...], preferred_element_type=jnp.float32)
    o_ref[...] = _prelu(o + b_ref[...], a_ref[0, 0])


def _tc_pre(x, w, b, a):
    return pl.pallas_call(
        _tc_pre_body,
        out_shape=jax.ShapeDtypeStruct((_N, _H), jnp.float32),
    )(x, w, b.reshape(1, _H), a.reshape(1, 1))


def _tc_layer_body(p_ref, x1_ref, relw_ref, relb_ref, rootw_ref, a_ref, o_ref):
    agg = p_ref[0] + p_ref[1]
    o = (jnp.dot(agg, relw_ref[...], preferred_element_type=jnp.float32)
         + relb_ref[...]
         + jnp.dot(x1_ref[...], rootw_ref[...],
                   preferred_element_type=jnp.float32))
    o_ref[...] = _prelu(o, a_ref[0, 0])


def _tc_layer(parts, x1, relw, relb, rootw, a):
    return pl.pallas_call(
        _tc_layer_body,
        out_shape=jax.ShapeDtypeStruct((_N, _H), jnp.float32),
    )(parts.reshape(2, _N, _H), x1, relw, relb.reshape(1, _H), rootw,
      a.reshape(1, 1))


def _tc_post_body(x_ref, x1_ref, postw_ref, postb_ref, posta_ref,
                  finwx_ref, finwh_ref, finb_ref, fina_ref, batch_ref, o_ref):
    h = _prelu(jnp.dot(x1_ref[...], postw_ref[...],
                       preferred_element_type=jnp.float32) + postb_ref[...],
               posta_ref[0, 0])
    f = (jnp.dot(x_ref[...], finwx_ref[...],
                 preferred_element_type=jnp.float32)
         + jnp.dot(h, finwh_ref[...], preferred_element_type=jnp.float32)
         + finb_ref[0, 0])
    f = _prelu(f, fina_ref[0, 0]) + batch_ref[...]
    m = jnp.max(f)
    lse = jnp.log(jnp.sum(jnp.exp(f - m))) + m
    o_ref[...] = f - lse


def _tc_post(x, x1, postw, postb, posta, finw, finb, fina, batch):
    return pl.pallas_call(
        _tc_post_body,
        out_shape=jax.ShapeDtypeStruct((_N, 1), jnp.float32),
    )(x, x1, postw, postb.reshape(1, _H), posta.reshape(1, 1),
      finw[:_D], finw[_D:], finb.reshape(1, 1), fina.reshape(1, 1),
      batch.astype(jnp.float32).reshape(_N, 1))


def kernel(x, edge_index, edge_weights, batch, pre_W, pre_b, pre_a,
           rel_W0, rel_b0, root_W0, a0,
           rel_W1, rel_b1, root_W1, a1,
           rel_W2, rel_b2, root_W2, a2,
           post_W, post_b, post_a, fin_W, fin_b, fin_a):
    pad = _EP - _E
    spread = (jnp.arange(pad, dtype=jnp.int32) * 13) % _N
    src = jnp.concatenate([edge_index[0], spread])
    dst = jnp.concatenate([edge_index[1], spread])
    ew = jnp.concatenate([edge_weights, jnp.zeros((pad,), jnp.float32)])
    x1 = _tc_pre(x, pre_W, pre_b, pre_a)
    for relw, relb, rootw, a in ((rel_W0, rel_b0, root_W0, a0),
                                 (rel_W1, rel_b1, root_W1, a1),
                                 (rel_W2, rel_b2, root_W2, a2)):
        parts = _sc_scatter(x1, src, dst, ew)
        x1 = _tc_layer(parts, x1, relw, relb, rootw, a)
    out = _tc_post(x, x1, post_W, post_b, post_a, fin_W, fin_b, fin_a, batch)
    return out.reshape(1, _NUM_CLASSES)
